# baseline (speedup 1.0000x reference)
import jax
import jax.numpy as jnp
from jax import lax
from jax.experimental import pallas as pl
from jax.experimental.pallas import tpu as pltpu
from jax.experimental.pallas import tpu_sc as plsc

N_NODES = 100000
D = 128
B = 10000
S = 16
L = 16
NC, NS = 2, 16
NW = NC * NS
CN = 8
CPW = 40
NB = 4
BPAD = NW * CPW * CN


def _body(table_hbm, idx_hbm, out_hbm, idx_v, rows_v, out_v, s0, s1, s2, s3):
    gsems = (s0, s1, s2, s3)
    wid = lax.axis_index("s") * NC + lax.axis_index("c")
    cbase = wid * CPW

    pltpu.sync_copy(idx_hbm.at[pl.ds(cbase, CPW)], idx_v)

    def issue(c, b):
        pltpu.async_copy(table_hbm.at[idx_v.at[c]], rows_v.at[b], gsems[b])

    def drain(b):
        pltpu.make_async_copy(table_hbm.at[idx_v.at[0]], rows_v.at[b],
                              gsems[b]).wait()

    for b in range(NB):
        issue(b, b)

    inv = jnp.full((L,), 1.0 / S, dtype=jnp.float32)

    def group_step(g, carry):
        for b in range(NB):
            c = g * NB + b
            drain(b)

            def node_step(i, carry2):
                for j in range(D // L):
                    acc = rows_v[b, i * S, pl.ds(j * L, L)]
                    for s in range(1, S):
                        acc = acc + rows_v[b, i * S + s, pl.ds(j * L, L)]
                    out_v[c * CN + i, pl.ds(j * L, L)] = acc * inv
                return carry2

            lax.fori_loop(0, CN, node_step, 0)

            @pl.when(g < CPW // NB - 1)
            def _():
                issue(c + NB, b)
        return carry

    lax.fori_loop(0, CPW // NB, group_step, 0)

    pltpu.sync_copy(out_v, out_hbm.at[pl.ds(wid * CPW * CN, CPW * CN)])


@jax.jit
def _sc_mean_agg(table, idx_pad):
    mesh = plsc.VectorSubcoreMesh(core_axis_name="c", subcore_axis_name="s")
    kfn = pl.kernel(
        _body,
        mesh=mesh,
        out_type=jax.ShapeDtypeStruct((BPAD, D), jnp.float32),
        scratch_types=[
            pltpu.VMEM((CPW, CN * S), jnp.int32),
            pltpu.VMEM((NB, CN * S, D), jnp.float32),
            pltpu.VMEM((CPW * CN, D), jnp.float32),
            pltpu.SemaphoreType.DMA,
            pltpu.SemaphoreType.DMA,
            pltpu.SemaphoreType.DMA,
            pltpu.SemaphoreType.DMA,
        ],
    )
    return kfn(table, idx_pad)


def kernel(features_weight, nodes, neigh_idx):
    idx2d = neigh_idx.astype(jnp.int32).reshape(B // CN, CN * S)
    idx_pad = jnp.pad(idx2d, ((0, NW * CPW - B // CN), (0, 0)))
    out = _sc_mean_agg(features_weight, idx_pad)
    return out[:B]


# speedup vs baseline: 3.7575x; 3.7575x over previous
import jax
import jax.numpy as jnp
from jax import lax
from jax.experimental import pallas as pl
from jax.experimental.pallas import tpu as pltpu
from jax.experimental.pallas import tpu_sc as plsc

N_NODES = 100000
D = 128
B = 10000
S = 16
L = 16
NC, NS = 2, 16
NW = NC * NS
CN = 8
CPW = 40
NB = 4
BPAD = NW * CPW * CN


def _body(table_hbm, idx_hbm, out_hbm, idx_v, rows_v, out_v, s0, s1, s2, s3):
    gsems = (s0, s1, s2, s3)
    wid = lax.axis_index("s") * NC + lax.axis_index("c")
    cbase = wid * CPW

    pltpu.sync_copy(idx_hbm.at[pl.ds(cbase, CPW)], idx_v)

    def issue(c, b):
        pltpu.async_copy(table_hbm.at[idx_v.at[c]], rows_v.at[b], gsems[b])

    def drain(b):
        pltpu.make_async_copy(table_hbm.at[idx_v.at[0]], rows_v.at[b],
                              gsems[b]).wait()

    for b in range(NB):
        issue(b, b)

    inv = jnp.full((L,), 1.0 / S, dtype=jnp.float32)

    def group_step(g, carry):
        for b in range(NB):
            c = g * NB + b
            drain(b)

            def node_step(i, carry2):
                for u in range(2):
                    node = i * 2 + u
                    base = node * S
                    acc = [rows_v[b, base, pl.ds(j * L, L)]
                           for j in range(D // L)]
                    for s in range(1, S):
                        for j in range(D // L):
                            acc[j] = acc[j] + rows_v[b, base + s,
                                                     pl.ds(j * L, L)]
                    for j in range(D // L):
                        out_v[c * CN + node, pl.ds(j * L, L)] = acc[j] * inv
                return carry2

            lax.fori_loop(0, CN // 2, node_step, 0)

            @pl.when(g < CPW // NB - 1)
            def _():
                issue(c + NB, b)
        return carry

    lax.fori_loop(0, CPW // NB, group_step, 0)

    pltpu.sync_copy(out_v, out_hbm.at[pl.ds(wid * CPW * CN, CPW * CN)])


@jax.jit
def _sc_mean_agg(table, idx_pad):
    mesh = plsc.VectorSubcoreMesh(core_axis_name="c", subcore_axis_name="s")
    kfn = pl.kernel(
        _body,
        mesh=mesh,
        out_type=jax.ShapeDtypeStruct((BPAD, D), jnp.float32),
        scratch_types=[
            pltpu.VMEM((CPW, CN * S), jnp.int32),
            pltpu.VMEM((NB, CN * S, D), jnp.float32),
            pltpu.VMEM((CPW * CN, D), jnp.float32),
            pltpu.SemaphoreType.DMA,
            pltpu.SemaphoreType.DMA,
            pltpu.SemaphoreType.DMA,
            pltpu.SemaphoreType.DMA,
        ],
    )
    return kfn(table, idx_pad)


def kernel(features_weight, nodes, neigh_idx):
    idx2d = neigh_idx.astype(jnp.int32).reshape(B // CN, CN * S)
    npad = NW * CPW - B // CN
    pad_ids = (jnp.arange(npad * CN * S, dtype=jnp.int32) * 613) % N_NODES
    idx_pad = jnp.concatenate([idx2d, pad_ids.reshape(npad, CN * S)], axis=0)
    out = _sc_mean_agg(features_weight, idx_pad)
    return out[:B]


# speedup vs baseline: 4.0355x; 1.0740x over previous
import jax
import jax.numpy as jnp
from jax import lax
from jax.experimental import pallas as pl
from jax.experimental.pallas import tpu as pltpu
from jax.experimental.pallas import tpu_sc as plsc

N_NODES = 100000
D = 128
B = 10000
S = 16
L = 16
NC, NS = 2, 16
NW = NC * NS
CN = 8
CPW = 40
NB = 4
NCHUNKS = B // CN


def _body(table_hbm, idx_hbm, out_hbm, idx_v, rows_v, out_v, s0, s1, s2, s3):
    gsems = (s0, s1, s2, s3)
    wid = lax.axis_index("s") * NC + lax.axis_index("c")
    last = wid == NW - 1
    cbase = pl.multiple_of(wid * CPW, CPW)
    ng = jnp.where(last, 3, CPW // NB)

    pltpu.sync_copy(idx_hbm.at[pl.ds(cbase, CPW)], idx_v)

    def issue(c, b):
        pltpu.async_copy(table_hbm.at[idx_v.at[c]], rows_v.at[b], gsems[b])

    def drain(b):
        pltpu.make_async_copy(table_hbm.at[idx_v.at[0]], rows_v.at[b],
                              gsems[b]).wait()

    for b in range(NB):
        issue(b, b)

    inv = jnp.full((L,), 1.0 / S, dtype=jnp.float32)

    def group_step(g, carry):
        for b in range(NB):
            cl = g * NB + b
            drain(b)

            def node_step(i, carry2):
                for u in range(2):
                    node = i * 2 + u
                    base = node * S
                    acc = [rows_v[b, base, pl.ds(j * L, L)]
                           for j in range(D // L)]
                    for s in range(1, S):
                        for j in range(D // L):
                            acc[j] = acc[j] + rows_v[b, base + s,
                                                     pl.ds(j * L, L)]
                    for j in range(D // L):
                        out_v[cl * CN + node, pl.ds(j * L, L)] = acc[j] * inv
                return carry2

            lax.fori_loop(0, CN // 2, node_step, 0)

            @pl.when(g < ng - 1)
            def _():
                issue(cl + NB, b)
        return carry

    lax.fori_loop(0, ng, group_step, 0)

    obase = pl.multiple_of(cbase * CN, CPW * CN)

    @pl.when(jnp.logical_not(last))
    def _():
        pltpu.sync_copy(out_v, out_hbm.at[pl.ds(obase, CPW * CN)])

    @pl.when(last)
    def _():
        rem = B - (NW - 1) * CPW * CN
        pltpu.sync_copy(out_v.at[pl.ds(0, rem)],
                        out_hbm.at[pl.ds(obase, rem)])


@jax.jit
def _sc_mean_agg(table, idx_pad):
    mesh = plsc.VectorSubcoreMesh(core_axis_name="c", subcore_axis_name="s")
    kfn = pl.kernel(
        _body,
        mesh=mesh,
        out_type=jax.ShapeDtypeStruct((B, D), jnp.float32),
        scratch_types=[
            pltpu.VMEM((CPW, CN * S), jnp.int32),
            pltpu.VMEM((NB, CN * S, D), jnp.float32),
            pltpu.VMEM((CPW * CN, D), jnp.float32),
            pltpu.SemaphoreType.DMA,
            pltpu.SemaphoreType.DMA,
            pltpu.SemaphoreType.DMA,
            pltpu.SemaphoreType.DMA,
        ],
    )
    return kfn(table, idx_pad)


def kernel(features_weight, nodes, neigh_idx):
    idx2d = neigh_idx.astype(jnp.int32).reshape(B // CN, CN * S)
    npad = NW * CPW - B // CN
    pad_ids = (jnp.arange(npad * CN * S, dtype=jnp.int32) * 613) % N_NODES
    idx_pad = jnp.concatenate([idx2d, pad_ids.reshape(npad, CN * S)], axis=0)
    return _sc_mean_agg(features_weight, idx_pad)


# speedup vs baseline: 4.0859x; 1.0125x over previous
import jax
import jax.numpy as jnp
from jax import lax
from jax.experimental import pallas as pl
from jax.experimental.pallas import tpu as pltpu
from jax.experimental.pallas import tpu_sc as plsc

N_NODES = 100000
D = 128
B = 10000
S = 16
L = 16
NC, NS = 2, 16
NW = NC * NS
CN = 8
CPW = 40
NB = 4
NPW = CPW * CN
NG_LAST = 3


def _body(table_hbm, idx_hbm, out_hbm, idx_raw, idx_v, rows_v, out_b,
          s0, s1, s2, s3, o0, o1, o2, o3):
    gsems = (s0, s1, s2, s3)
    osems = (o0, o1, o2, o3)
    wid = lax.axis_index("s") * NC + lax.axis_index("c")
    last = wid == NW - 1
    nbase = pl.multiple_of(jnp.where(last, B - NPW, wid * NPW), 8)
    ls = jnp.where(last, CPW - NG_LAST * NB, 0)
    ng = jnp.where(last, NG_LAST, CPW // NB)
    obase = pl.multiple_of(jnp.where(last, B - NG_LAST * NB * CN,
                                     wid * NPW), 8)

    pltpu.sync_copy(idx_hbm.at[pl.ds(nbase, NPW)], idx_raw)

    def repack_step(c, carry):
        for n in range(CN):
            idx_v[c, pl.ds(n * S, S)] = idx_raw[c * CN + n, :]
        return carry

    lax.fori_loop(0, CPW, repack_step, 0)

    def issue(c, b):
        pltpu.async_copy(table_hbm.at[idx_v.at[c]], rows_v.at[b], gsems[b])

    def drain(b):
        pltpu.make_async_copy(table_hbm.at[idx_v.at[0]], rows_v.at[b],
                              gsems[b]).wait()

    def owait(b):
        pltpu.make_async_copy(out_b.at[b],
                              out_hbm.at[pl.ds(0, CN)], osems[b]).wait()

    for b in range(NB):
        issue(ls + b, b)

    inv = jnp.full((L,), 1.0 / S, dtype=jnp.float32)

    def group_step(g, carry):
        for b in range(NB):
            cl = g * NB + b
            drain(b)

            @pl.when(g > 0)
            def _():
                owait(b)

            def node_step(i, carry2):
                for u in range(2):
                    node = i * 2 + u
                    base = node * S
                    acc = [rows_v[b, base, pl.ds(j * L, L)]
                           for j in range(D // L)]
                    for s in range(1, S):
                        for j in range(D // L):
                            acc[j] = acc[j] + rows_v[b, base + s,
                                                     pl.ds(j * L, L)]
                    for j in range(D // L):
                        out_b[b, node, pl.ds(j * L, L)] = acc[j] * inv
                return carry2

            lax.fori_loop(0, CN // 2, node_step, 0)

            row = pl.multiple_of(obase + cl * CN, 8)
            pltpu.async_copy(out_b.at[b], out_hbm.at[pl.ds(row, CN)],
                             osems[b])

            @pl.when(g < ng - 1)
            def _():
                issue(ls + cl + NB, b)
        return carry

    lax.fori_loop(0, ng, group_step, 0)

    for b in range(NB):
        owait(b)


@jax.jit
def _sc_mean_agg(table, idx):
    mesh = plsc.VectorSubcoreMesh(core_axis_name="c", subcore_axis_name="s")
    kfn = pl.kernel(
        _body,
        mesh=mesh,
        out_type=jax.ShapeDtypeStruct((B, D), jnp.float32),
        scratch_types=[
            pltpu.VMEM((NPW, S), jnp.int32),
            pltpu.VMEM((CPW, CN * S), jnp.int32),
            pltpu.VMEM((NB, CN * S, D), jnp.float32),
            pltpu.VMEM((NB, CN, D), jnp.float32),
            pltpu.SemaphoreType.DMA,
            pltpu.SemaphoreType.DMA,
            pltpu.SemaphoreType.DMA,
            pltpu.SemaphoreType.DMA,
            pltpu.SemaphoreType.DMA,
            pltpu.SemaphoreType.DMA,
            pltpu.SemaphoreType.DMA,
            pltpu.SemaphoreType.DMA,
        ],
    )
    return kfn(table, idx)


def kernel(features_weight, nodes, neigh_idx):
    return _sc_mean_agg(features_weight, neigh_idx.astype(jnp.int32))


# speedup vs baseline: 4.1943x; 1.0265x over previous
import jax
import jax.numpy as jnp
from jax import lax
from jax.experimental import pallas as pl
from jax.experimental.pallas import tpu as pltpu
from jax.experimental.pallas import tpu_sc as plsc

N_NODES = 100000
D = 128
B = 10000
S = 16
L = 16
NC, NS = 2, 16
NW = NC * NS
CN = 8
CPW = 40
NB = 4
NPW = CPW * CN
NG_LAST = 3


def _body(table_hbm, idx_hbm, out_hbm, idx_raw, idx_v, rows_v, out_b,
          s0, s1, s2, s3, o0, o1, o2, o3):
    gsems = (s0, s1, s2, s3)
    osems = (o0, o1, o2, o3)
    wid = lax.axis_index("s") * NC + lax.axis_index("c")
    last = wid == NW - 1
    nbase = pl.multiple_of(jnp.where(last, B - NPW, wid * NPW), 8)
    ls = jnp.where(last, CPW - NG_LAST * NB, 0)
    ng = jnp.where(last, NG_LAST, CPW // NB)
    obase = pl.multiple_of(jnp.where(last, B - NG_LAST * NB * CN,
                                     wid * NPW), 8)

    pltpu.sync_copy(idx_hbm.at[pl.ds(nbase, NPW)], idx_raw)

    def repack_step(c, carry):
        for n in range(CN):
            idx_v[c, pl.ds(n * S, S)] = idx_raw[c * CN + n, :]
        return carry

    lax.fori_loop(0, CPW, repack_step, 0)

    def issue(c, b):
        pltpu.async_copy(table_hbm.at[idx_v.at[c]], rows_v.at[b], gsems[b])

    def drain(b):
        pltpu.make_async_copy(table_hbm.at[idx_v.at[0]], rows_v.at[b],
                              gsems[b]).wait()

    def owait(b):
        pltpu.make_async_copy(out_b.at[b],
                              out_hbm.at[pl.ds(0, CN)], osems[b]).wait()

    for b in range(NB):
        issue(ls + b, b)

    inv = jnp.full((L,), 1.0 / S, dtype=jnp.float32)

    def group_step(g, carry):
        for b in range(NB):
            cl = g * NB + b
            drain(b)

            @pl.when(g > 0)
            def _():
                owait(b)

            def node_step(node, carry2):
                base = node * S
                acc = [rows_v[b, base, pl.ds(j * L, L)]
                       for j in range(D // L)]
                for s in range(1, S):
                    for j in range(D // L):
                        acc[j] = acc[j] + rows_v[b, base + s,
                                                 pl.ds(j * L, L)]
                for j in range(D // L):
                    out_b[b, node, pl.ds(j * L, L)] = acc[j] * inv
                return carry2

            lax.fori_loop(0, CN, node_step, 0)

            row = pl.multiple_of(obase + cl * CN, 8)
            pltpu.async_copy(out_b.at[b], out_hbm.at[pl.ds(row, CN)],
                             osems[b])

            @pl.when(g < ng - 1)
            def _():
                issue(ls + cl + NB, b)
        return carry

    lax.fori_loop(0, ng, group_step, 0)

    for b in range(NB):
        owait(b)


@jax.jit
def _sc_mean_agg(table, idx):
    mesh = plsc.VectorSubcoreMesh(core_axis_name="c", subcore_axis_name="s")
    kfn = pl.kernel(
        _body,
        mesh=mesh,
        out_type=jax.ShapeDtypeStruct((B, D), jnp.float32),
        scratch_types=[
            pltpu.VMEM((NPW, S), jnp.int32),
            pltpu.VMEM((CPW, CN * S), jnp.int32),
            pltpu.VMEM((NB, CN * S, D), jnp.float32),
            pltpu.VMEM((NB, CN, D), jnp.float32),
            pltpu.SemaphoreType.DMA,
            pltpu.SemaphoreType.DMA,
            pltpu.SemaphoreType.DMA,
            pltpu.SemaphoreType.DMA,
            pltpu.SemaphoreType.DMA,
            pltpu.SemaphoreType.DMA,
            pltpu.SemaphoreType.DMA,
            pltpu.SemaphoreType.DMA,
        ],
    )
    return kfn(table, idx)


def kernel(features_weight, nodes, neigh_idx):
    return _sc_mean_agg(features_weight, neigh_idx.astype(jnp.int32))


# speedup vs baseline: 4.2669x; 1.0173x over previous
import jax
import jax.numpy as jnp
from jax import lax
from jax.experimental import pallas as pl
from jax.experimental.pallas import tpu as pltpu
from jax.experimental.pallas import tpu_sc as plsc

N_NODES = 100000
D = 128
B = 10000
S = 16
L = 16
NC, NS = 2, 16
NW = NC * NS
CN = 8
CPW = 40
NB = 4
NPW = CPW * CN
NG_LAST = 3


def _body(table_hbm, idx_hbm, out_hbm, idx_raw, idx_v, rows_v, out_b,
          gsem, osem):
    wid = lax.axis_index("s") * NC + lax.axis_index("c")
    last = wid == NW - 1
    nbase = pl.multiple_of(jnp.where(last, B - NPW, wid * NPW), 8)
    ls = jnp.where(last, CPW - NG_LAST * NB, 0)
    ng = jnp.where(last, NG_LAST, CPW // NB)
    obase = pl.multiple_of(jnp.where(last, B - NG_LAST * NB * CN,
                                     wid * NPW), 8)

    pltpu.sync_copy(idx_hbm.at[pl.ds(nbase, NPW)], idx_raw)

    def repack_step(c, carry):
        for n in range(CN):
            idx_v[c, pl.ds(n * S, S)] = idx_raw[c * CN + n, :]
        return carry

    lax.fori_loop(0, CPW, repack_step, 0)

    def issue(c, b):
        pltpu.async_copy(table_hbm.at[idx_v.at[c]], rows_v.at[b], gsem)

    def drain(b):
        pltpu.make_async_copy(table_hbm.at[idx_v.at[0]], rows_v.at[b],
                              gsem).wait()

    def owait(b):
        pltpu.make_async_copy(out_b.at[b],
                              out_hbm.at[pl.ds(0, CN)], osem).wait()

    for b in range(NB):
        issue(ls + b, b)

    inv = jnp.full((L,), 1.0 / S, dtype=jnp.float32)

    def group_step(g, carry):
        for b in range(NB):
            cl = g * NB + b
            drain(b)

            @pl.when(g > 0)
            def _():
                owait(b)

            def node_step(node, carry2):
                base = node * S
                acc = [rows_v[b, base, pl.ds(j * L, L)]
                       for j in range(D // L)]
                for s in range(1, S):
                    for j in range(D // L):
                        acc[j] = acc[j] + rows_v[b, base + s,
                                                 pl.ds(j * L, L)]
                for j in range(D // L):
                    out_b[b, node, pl.ds(j * L, L)] = acc[j] * inv
                return carry2

            lax.fori_loop(0, CN, node_step, 0)

            row = pl.multiple_of(obase + cl * CN, 8)
            pltpu.async_copy(out_b.at[b], out_hbm.at[pl.ds(row, CN)], osem)

            @pl.when(g < ng - 1)
            def _():
                issue(ls + cl + NB, b)
        return carry

    lax.fori_loop(0, ng, group_step, 0)

    for b in range(NB):
        owait(b)


@jax.jit
def _sc_mean_agg(table, idx):
    mesh = plsc.VectorSubcoreMesh(core_axis_name="c", subcore_axis_name="s")
    kfn = pl.kernel(
        _body,
        mesh=mesh,
        out_type=jax.ShapeDtypeStruct((B, D), jnp.float32),
        scratch_types=[
            pltpu.VMEM((NPW, S), jnp.int32),
            pltpu.VMEM((CPW, CN * S), jnp.int32),
            pltpu.VMEM((NB, CN * S, D), jnp.float32),
            pltpu.VMEM((NB, CN, D), jnp.float32),
            pltpu.SemaphoreType.DMA,
            pltpu.SemaphoreType.DMA,
        ],
    )
    return kfn(table, idx)


def kernel(features_weight, nodes, neigh_idx):
    return _sc_mean_agg(features_weight, neigh_idx.astype(jnp.int32))
